# baseline (speedup 1.0000x reference)
import functools

import jax
import jax.numpy as jnp
import numpy as np
from jax.experimental import pallas as pl

_B = 8
_P = 2048
_ATTR = 3
_OUT = 128
_RATIOS = (0.25, 0.25)
_KNN = (32, 64)
_BN_SCALE = 1.0 / np.sqrt(1.0 + 1e-5)


def _mlp_kernel(*refs, n_layers):
    h = refs[0][...]
    for i in range(n_layers):
        W = refs[1 + 2 * i][...]
        b = refs[2 + 2 * i][...]
        h = jnp.dot(h, W, preferred_element_type=jnp.float32) + b[None, :]
        if i < n_layers - 1:
            h = jax.nn.relu(h * _BN_SCALE)
    refs[-1][...] = h


def _mlp_pallas(params, h, blk=1024):
    n, din = h.shape
    dout = params[-1][0].shape[1]
    n_layers = len(params)
    npad = -n % blk
    if npad:
        h = jnp.pad(h, ((0, npad), (0, 0)))
    grid = (h.shape[0] // blk,)
    in_specs = [pl.BlockSpec((blk, din), lambda i: (i, 0))]
    args = [h]
    for (W, b) in params:
        in_specs.append(pl.BlockSpec(W.shape, lambda i: (0, 0)))
        in_specs.append(pl.BlockSpec(b.shape, lambda i: (0,)))
        args.extend([W, b])
    out = pl.pallas_call(
        functools.partial(_mlp_kernel, n_layers=n_layers),
        grid=grid,
        in_specs=in_specs,
        out_specs=pl.BlockSpec((blk, dout), lambda i: (i, 0)),
        out_shape=jax.ShapeDtypeStruct((h.shape[0], dout), jnp.float32),
    )(*args)
    return out[:n]


def _fps_indices(pos_b, n_sample):
    Pn = pos_b.shape[0]

    def body(i, st):
        dists, idxs = st
        last = idxs[i - 1]
        d = jnp.sum((pos_b - pos_b[last]) ** 2, axis=-1)
        dists = jnp.minimum(dists, d)
        idxs = idxs.at[i].set(jnp.argmax(dists).astype(jnp.int32))
        return (dists, idxs)

    dists0 = jnp.full((Pn,), jnp.inf, dtype=pos_b.dtype)
    idxs0 = jnp.zeros((n_sample,), dtype=jnp.int32)
    _, idxs = jax.lax.fori_loop(1, n_sample, body, (dists0, idxs0))
    return idxs


def _sa_layer(xb, posb, ratio, k, mlp_p, lin_p):
    Bn, Pn, _ = posb.shape
    n_s = int(Pn * ratio)
    idx = jax.vmap(lambda p: _fps_indices(p, n_s))(posb)
    posq = jnp.take_along_axis(posb, idx[..., None], axis=1)
    xdst = jnp.take_along_axis(xb, idx[..., None], axis=1)

    def nbrs(ps, pq):
        d2 = jnp.sum((pq[:, None, :] - ps[None, :, :]) ** 2, axis=-1)
        _, nidx = jax.lax.top_k(-d2, k)
        return nidx

    nidx = jax.vmap(nbrs)(posb, posq)
    x_j = jax.vmap(lambda a, i_: a[i_])(xb, nidx)
    p_j = jax.vmap(lambda a, i_: a[i_])(posb, nidx)
    rel = p_j - posq[:, :, None, :]
    msg = jnp.concatenate([x_j, rel], axis=-1)
    din = msg.shape[-1]
    h = _mlp_pallas(mlp_p, msg.reshape(-1, din)).reshape(Bn, n_s, k, -1)
    agg = jnp.max(h, axis=2)
    Wl, bl = lin_p
    return agg + xdst @ Wl + bl, posq


def _knn_interp(xs, ps, pd, k):
    def one(xs_, ps_, pd_):
        d2 = jnp.sum((pd_[:, None, :] - ps_[None, :, :]) ** 2, axis=-1)
        _, idx = jax.lax.top_k(-d2, k)
        diff = ps_[idx] - pd_[:, None, :]
        d2k = jnp.sum(diff * diff, axis=-1)
        w = 1.0 / jnp.maximum(d2k, 1e-16)
        feats = xs_[idx]
        return jnp.sum(feats * w[..., None], axis=1) / jnp.sum(w, axis=1, keepdims=True)

    return jax.vmap(one)(xs, ps, pd)


def kernel(x, pos, batch, params):
    xb = x.reshape(_B, _P, _ATTR)
    posb = pos.reshape(_B, _P, 3)
    x1, pos1 = _sa_layer(xb, posb, _RATIOS[0], _KNN[0], params["sa0_mlp"], params["sa0_lin"])
    x2, pos2 = _sa_layer(x1, pos1, _RATIOS[1], _KNN[1], params["sa1_mlp"], params["sa1_lin"])
    n2 = pos2.shape[1]
    h = _mlp_pallas(params["glob_mlp"], jnp.concatenate([x2, pos2], axis=-1).reshape(-1, 256 + 3))
    xg = jnp.max(h.reshape(_B, n2, 512), axis=1)
    posg = jnp.zeros((_B, 1, 3), dtype=pos.dtype)
    xi = _knn_interp(xg[:, None, :], posg, pos2, 1)
    h = jnp.concatenate([xi, x2], axis=-1)
    h = _mlp_pallas(params["fp0_mlp"], h.reshape(-1, 512 + 256)).reshape(_B, n2, 256)
    xi = _knn_interp(h, pos2, pos1, 3)
    h = jnp.concatenate([xi, x1], axis=-1)
    h = _mlp_pallas(params["fp1_mlp"], h.reshape(-1, 256 + 128)).reshape(_B, pos1.shape[1], 128)
    xi = _knn_interp(h, pos1, posb, 3)
    h = jnp.concatenate([xi, xb], axis=-1)
    h = _mlp_pallas(params["fp2_mlp"], h.reshape(-1, 128 + _ATTR))
    return h.reshape(_B * _P, _OUT)


# speedup vs baseline: 1.3826x; 1.3826x over previous
import functools

import jax
import jax.numpy as jnp
import numpy as np
from jax.experimental import pallas as pl

_B = 8
_P = 2048
_ATTR = 3
_OUT = 128
_RATIOS = (0.25, 0.25)
_KNN = (32, 64)
_BN_SCALE = 1.0 / np.sqrt(1.0 + 1e-5)


def _mlp_kernel(*refs, n_layers):
    h = refs[0][...]
    for i in range(n_layers):
        W = refs[1 + 2 * i][...]
        b = refs[2 + 2 * i][...]
        h = jnp.dot(h, W, preferred_element_type=jnp.float32) + b[None, :]
        if i < n_layers - 1:
            h = jax.nn.relu(h * _BN_SCALE)
    refs[-1][...] = h


def _mlp_pallas(params, h, blk=1024):
    n, din = h.shape
    dout = params[-1][0].shape[1]
    n_layers = len(params)
    npad = -n % blk
    if npad:
        h = jnp.pad(h, ((0, npad), (0, 0)))
    grid = (h.shape[0] // blk,)
    in_specs = [pl.BlockSpec((blk, din), lambda i: (i, 0))]
    args = [h]
    for (W, b) in params:
        in_specs.append(pl.BlockSpec(W.shape, lambda i: (0, 0)))
        in_specs.append(pl.BlockSpec(b.shape, lambda i: (0,)))
        args.extend([W, b])
    out = pl.pallas_call(
        functools.partial(_mlp_kernel, n_layers=n_layers),
        grid=grid,
        in_specs=in_specs,
        out_specs=pl.BlockSpec((blk, dout), lambda i: (i, 0)),
        out_shape=jax.ShapeDtypeStruct((h.shape[0], dout), jnp.float32),
    )(*args)
    return out[:n]


def _fps_level(px, py, pz, n_sample, attrs=()):
    Bn, Pn = px.shape
    iota = jax.lax.broadcasted_iota(jnp.int32, (Bn, Pn), 1)
    iota_ns = jax.lax.broadcasted_iota(jnp.int32, (Bn, n_sample), 1)
    first = iota_ns == 0
    q0 = (px[:, 0:1], py[:, 0:1], pz[:, 0:1])
    idx_acc0 = jnp.zeros((Bn, n_sample), dtype=jnp.int32)
    pos_acc0 = tuple(jnp.where(first, q0[c], 0.0) for c in range(3))
    attr_acc0 = tuple(jnp.where(first, a[:, 0:1], 0.0) for a in attrs)

    def body(i, st):
        dists, qx, qy, qz, idx_acc, pos_acc, attr_acc = st
        d = (px - qx) ** 2 + (py - qy) ** 2 + (pz - qz) ** 2
        dists = jnp.minimum(dists, d)
        m = jnp.max(dists, axis=1, keepdims=True)
        sel = jnp.min(jnp.where(dists == m, iota, Pn), axis=1, keepdims=True)
        here = iota_ns == i
        idx_acc = jnp.where(here, sel, idx_acc)
        oh = iota == sel
        nq = tuple(jnp.sum(jnp.where(oh, p, 0.0), axis=1, keepdims=True)
                   for p in (px, py, pz))
        pos_acc = tuple(jnp.where(here, nq[c], pos_acc[c]) for c in range(3))
        attr_acc = tuple(
            jnp.where(here,
                      jnp.sum(jnp.where(oh, a, 0.0), axis=1, keepdims=True),
                      acc)
            for a, acc in zip(attrs, attr_acc))
        return (dists, nq[0], nq[1], nq[2], idx_acc, pos_acc, attr_acc)

    dists0 = jnp.full(px.shape, jnp.inf, dtype=jnp.float32)
    st = jax.lax.fori_loop(
        1, n_sample, body,
        (dists0,) + q0 + (idx_acc0, pos_acc0, attr_acc0))
    return st[4], st[5], st[6]


def _fps2_kernel(posT_ref, xT_ref, idx0_ref, pos1T_ref, xdst1T_ref,
                 idx1_ref, pos2T_ref):
    px, py, pz = posT_ref[0], posT_ref[1], posT_ref[2]
    ax, ay, az = xT_ref[0], xT_ref[1], xT_ref[2]
    idx0, pos1, xdst1 = _fps_level(px, py, pz, _P // 4, (ax, ay, az))
    idx0_ref[...] = idx0
    for c in range(3):
        pos1T_ref[c] = pos1[c]
        xdst1T_ref[c] = xdst1[c]
    idx1, pos2, _ = _fps_level(pos1[0], pos1[1], pos1[2], _P // 16)
    idx1_ref[...] = idx1
    for c in range(3):
        pos2T_ref[c] = pos2[c]


def _fps2(posb, xb):
    posT = posb.transpose(2, 0, 1)
    xT = xb.transpose(2, 0, 1)
    n1, n2 = _P // 4, _P // 16
    out = pl.pallas_call(
        _fps2_kernel,
        out_shape=(
            jax.ShapeDtypeStruct((_B, n1), jnp.int32),
            jax.ShapeDtypeStruct((3, _B, n1), jnp.float32),
            jax.ShapeDtypeStruct((3, _B, n1), jnp.float32),
            jax.ShapeDtypeStruct((_B, n2), jnp.int32),
            jax.ShapeDtypeStruct((3, _B, n2), jnp.float32),
        ),
    )(posT, xT)
    idx0, pos1T, xdst1T, idx1, pos2T = out
    return (idx0, pos1T.transpose(1, 2, 0), xdst1T.transpose(1, 2, 0),
            idx1, pos2T.transpose(1, 2, 0))


def _sa_layer(xb, posb, posq, xdst, k, mlp_p, lin_p):
    Bn = posb.shape[0]
    n_s = posq.shape[1]

    def nbrs(ps, pq):
        d2 = jnp.sum((pq[:, None, :] - ps[None, :, :]) ** 2, axis=-1)
        _, nidx = jax.lax.top_k(-d2, k)
        return nidx

    nidx = jax.vmap(nbrs)(posb, posq)
    x_j = jax.vmap(lambda a, i_: a[i_])(xb, nidx)
    p_j = jax.vmap(lambda a, i_: a[i_])(posb, nidx)
    rel = p_j - posq[:, :, None, :]
    msg = jnp.concatenate([x_j, rel], axis=-1)
    din = msg.shape[-1]
    h = _mlp_pallas(mlp_p, msg.reshape(-1, din)).reshape(Bn, n_s, k, -1)
    agg = jnp.max(h, axis=2)
    Wl, bl = lin_p
    return agg + xdst @ Wl + bl


def _knn_interp(xs, ps, pd, k):
    def one(xs_, ps_, pd_):
        d2 = jnp.sum((pd_[:, None, :] - ps_[None, :, :]) ** 2, axis=-1)
        _, idx = jax.lax.top_k(-d2, k)
        diff = ps_[idx] - pd_[:, None, :]
        d2k = jnp.sum(diff * diff, axis=-1)
        w = 1.0 / jnp.maximum(d2k, 1e-16)
        feats = xs_[idx]
        return jnp.sum(feats * w[..., None], axis=1) / jnp.sum(w, axis=1, keepdims=True)

    return jax.vmap(one)(xs, ps, pd)


def kernel(x, pos, batch, params):
    xb = x.reshape(_B, _P, _ATTR)
    posb = pos.reshape(_B, _P, 3)
    idx0, pos1, xdst1, idx1, pos2 = _fps2(posb, xb)
    x1 = _sa_layer(xb, posb, pos1, xdst1, _KNN[0], params["sa0_mlp"], params["sa0_lin"])
    xdst2 = jnp.take_along_axis(x1, idx1[..., None], axis=1)
    x2 = _sa_layer(x1, pos1, pos2, xdst2, _KNN[1], params["sa1_mlp"], params["sa1_lin"])
    n2 = pos2.shape[1]
    h = _mlp_pallas(params["glob_mlp"], jnp.concatenate([x2, pos2], axis=-1).reshape(-1, 256 + 3))
    xg = jnp.max(h.reshape(_B, n2, 512), axis=1)
    posg = jnp.zeros((_B, 1, 3), dtype=pos.dtype)
    xi = _knn_interp(xg[:, None, :], posg, pos2, 1)
    h = jnp.concatenate([xi, x2], axis=-1)
    h = _mlp_pallas(params["fp0_mlp"], h.reshape(-1, 512 + 256)).reshape(_B, n2, 256)
    xi = _knn_interp(h, pos2, pos1, 3)
    h = jnp.concatenate([xi, x1], axis=-1)
    h = _mlp_pallas(params["fp1_mlp"], h.reshape(-1, 256 + 128)).reshape(_B, pos1.shape[1], 128)
    xi = _knn_interp(h, pos1, posb, 3)
    h = jnp.concatenate([xi, xb], axis=-1)
    h = _mlp_pallas(params["fp2_mlp"], h.reshape(-1, 128 + _ATTR))
    return h.reshape(_B * _P, _OUT)


# speedup vs baseline: 15.0832x; 10.9091x over previous
import functools

import jax
import jax.numpy as jnp
import numpy as np
from jax.experimental import pallas as pl

_B = 8
_P = 2048
_ATTR = 3
_OUT = 128
_RATIOS = (0.25, 0.25)
_KNN = (32, 64)
_BN_SCALE = 1.0 / np.sqrt(1.0 + 1e-5)


def _mlp_kernel(*refs, n_layers):
    h = refs[0][...]
    for i in range(n_layers):
        W = refs[1 + 2 * i][...]
        b = refs[2 + 2 * i][...]
        h = jnp.dot(h, W, preferred_element_type=jnp.float32) + b[None, :]
        if i < n_layers - 1:
            h = jax.nn.relu(h * _BN_SCALE)
    refs[-1][...] = h


def _mlp_pallas(params, h, blk=1024):
    n, din = h.shape
    dout = params[-1][0].shape[1]
    n_layers = len(params)
    npad = -n % blk
    if npad:
        h = jnp.pad(h, ((0, npad), (0, 0)))
    grid = (h.shape[0] // blk,)
    in_specs = [pl.BlockSpec((blk, din), lambda i: (i, 0))]
    args = [h]
    for (W, b) in params:
        in_specs.append(pl.BlockSpec(W.shape, lambda i: (0, 0)))
        in_specs.append(pl.BlockSpec(b.shape, lambda i: (0,)))
        args.extend([W, b])
    out = pl.pallas_call(
        functools.partial(_mlp_kernel, n_layers=n_layers),
        grid=grid,
        in_specs=in_specs,
        out_specs=pl.BlockSpec((blk, dout), lambda i: (i, 0)),
        out_shape=jax.ShapeDtypeStruct((h.shape[0], dout), jnp.float32),
    )(*args)
    return out[:n]


def _fps_level(px, py, pz, n_sample, attrs=()):
    Bn, Pn = px.shape
    iota = jax.lax.broadcasted_iota(jnp.int32, (Bn, Pn), 1)
    iota_ns = jax.lax.broadcasted_iota(jnp.int32, (Bn, n_sample), 1)
    first = iota_ns == 0
    q0 = (px[:, 0:1], py[:, 0:1], pz[:, 0:1])
    idx_acc0 = jnp.zeros((Bn, n_sample), dtype=jnp.int32)
    pos_acc0 = tuple(jnp.where(first, q0[c], 0.0) for c in range(3))
    attr_acc0 = tuple(jnp.where(first, a[:, 0:1], 0.0) for a in attrs)

    def body(i, st):
        dists, qx, qy, qz, idx_acc, pos_acc, attr_acc = st
        d = (px - qx) ** 2 + (py - qy) ** 2 + (pz - qz) ** 2
        dists = jnp.minimum(dists, d)
        m = jnp.max(dists, axis=1, keepdims=True)
        sel = jnp.min(jnp.where(dists == m, iota, Pn), axis=1, keepdims=True)
        here = iota_ns == i
        idx_acc = jnp.where(here, sel, idx_acc)
        oh = iota == sel
        nq = tuple(jnp.sum(jnp.where(oh, p, 0.0), axis=1, keepdims=True)
                   for p in (px, py, pz))
        pos_acc = tuple(jnp.where(here, nq[c], pos_acc[c]) for c in range(3))
        attr_acc = tuple(
            jnp.where(here,
                      jnp.sum(jnp.where(oh, a, 0.0), axis=1, keepdims=True),
                      acc)
            for a, acc in zip(attrs, attr_acc))
        return (dists, nq[0], nq[1], nq[2], idx_acc, pos_acc, attr_acc)

    dists0 = jnp.full(px.shape, jnp.inf, dtype=jnp.float32)
    st = jax.lax.fori_loop(
        1, n_sample, body,
        (dists0,) + q0 + (idx_acc0, pos_acc0, attr_acc0))
    return st[4], st[5], st[6]


def _fps2_kernel(posT_ref, xT_ref, idx0_ref, pos1T_ref, xdst1T_ref,
                 idx1_ref, pos2T_ref):
    px, py, pz = posT_ref[0], posT_ref[1], posT_ref[2]
    ax, ay, az = xT_ref[0], xT_ref[1], xT_ref[2]
    idx0, pos1, xdst1 = _fps_level(px, py, pz, _P // 4, (ax, ay, az))
    idx0_ref[...] = idx0
    for c in range(3):
        pos1T_ref[c] = pos1[c]
        xdst1T_ref[c] = xdst1[c]
    idx1, pos2, _ = _fps_level(pos1[0], pos1[1], pos1[2], _P // 16)
    idx1_ref[...] = idx1
    for c in range(3):
        pos2T_ref[c] = pos2[c]


def _fps2(posb, xb):
    posT = posb.transpose(2, 0, 1)
    xT = xb.transpose(2, 0, 1)
    n1, n2 = _P // 4, _P // 16
    out = pl.pallas_call(
        _fps2_kernel,
        out_shape=(
            jax.ShapeDtypeStruct((_B, n1), jnp.int32),
            jax.ShapeDtypeStruct((3, _B, n1), jnp.float32),
            jax.ShapeDtypeStruct((3, _B, n1), jnp.float32),
            jax.ShapeDtypeStruct((_B, n2), jnp.int32),
            jax.ShapeDtypeStruct((3, _B, n2), jnp.float32),
        ),
    )(posT, xT)
    idx0, pos1T, xdst1T, idx1, pos2T = out
    return (idx0, pos1T.transpose(1, 2, 0), xdst1T.transpose(1, 2, 0),
            idx1, pos2T.transpose(1, 2, 0))


def _d2_matrix(posq, posT):
    d2 = None
    for c in range(3):
        diff = posq[:, c:c + 1] - posT[c:c + 1, :]
        sq = diff * diff
        d2 = sq if d2 is None else d2 + sq
    return d2


def _sa_kernel(xpos_ref, posT_ref, posq_ref, xd_ref, W1_ref, b1_ref,
               W2_ref, b2_ref, Wl_ref, bl_ref, *refs,
               k, din_q, has_mid):
    xmid_ref = refs[0] if has_mid else None
    out_ref = refs[-1]
    xpos = xpos_ref[0]
    posT = posT_ref[0]
    posq = posq_ref[0]
    W1 = W1_ref[...]
    s1 = jnp.dot(xpos, W1, preferred_element_type=jnp.float32)
    Wp = W1[din_q:, :]
    qc = b1_ref[...][None, :] - jnp.dot(posq, Wp,
                                        preferred_element_type=jnp.float32)
    d2 = _d2_matrix(posq, posT)
    W2 = W2_ref[...]
    b2 = b2_ref[...][None, :]
    Q = d2.shape[0]
    agg0 = jnp.full((Q, W2.shape[1]), -jnp.inf, dtype=jnp.float32)

    def body(t, st):
        d2_, agg = st
        m = jnp.min(d2_, axis=1, keepdims=True)
        oh = d2_ == m
        d2_ = jnp.where(oh, jnp.inf, d2_)
        g = jnp.dot(oh.astype(jnp.float32), s1,
                    preferred_element_type=jnp.float32)
        h1 = jax.nn.relu((g + qc) * _BN_SCALE)
        h2 = jnp.dot(h1, W2, preferred_element_type=jnp.float32) + b2
        return d2_, jnp.maximum(agg, h2)

    _, agg = jax.lax.fori_loop(0, k, body, (d2, agg0))
    xd = xd_ref[0]
    if has_mid:
        xd = jnp.dot(xd, xmid_ref[0], preferred_element_type=jnp.float32)
    lin = jnp.dot(xd, Wl_ref[...], preferred_element_type=jnp.float32)
    out_ref[0] = agg + lin + bl_ref[...][None, :]


def _sa_layer(xpos, posT, posq, xd, xmid, k, mlp_p, lin_p):
    Bn, S, din = xpos.shape
    Q = posq.shape[1]
    (W1, b1), (W2, b2) = mlp_p
    Wl, bl = lin_p
    args = [xpos, posT, posq, xd, W1, b1, W2, b2, Wl, bl]
    in_specs = [
        pl.BlockSpec((1, S, din), lambda i: (i, 0, 0)),
        pl.BlockSpec((1, 3, S), lambda i: (i, 0, 0)),
        pl.BlockSpec((1, Q, 3), lambda i: (i, 0, 0)),
        pl.BlockSpec((1,) + xd.shape[1:], lambda i: (i, 0, 0)),
        pl.BlockSpec(W1.shape, lambda i: (0, 0)),
        pl.BlockSpec(b1.shape, lambda i: (0,)),
        pl.BlockSpec(W2.shape, lambda i: (0, 0)),
        pl.BlockSpec(b2.shape, lambda i: (0,)),
        pl.BlockSpec(Wl.shape, lambda i: (0, 0)),
        pl.BlockSpec(bl.shape, lambda i: (0,)),
    ]
    if xmid is not None:
        args.append(xmid)
        in_specs.append(pl.BlockSpec((1,) + xmid.shape[1:], lambda i: (i, 0, 0)))
    dout = W2.shape[1]
    return pl.pallas_call(
        functools.partial(_sa_kernel, k=k, din_q=din - 3,
                          has_mid=xmid is not None),
        grid=(Bn,),
        in_specs=in_specs,
        out_specs=pl.BlockSpec((1, Q, dout), lambda i: (i, 0, 0)),
        out_shape=jax.ShapeDtypeStruct((Bn, Q, dout), jnp.float32),
    )(*args)


def _glob_kernel(in_ref, W1_ref, b1_ref, W2_ref, b2_ref, out_ref):
    h = jnp.dot(in_ref[0], W1_ref[...],
                preferred_element_type=jnp.float32) + b1_ref[...][None, :]
    h = jax.nn.relu(h * _BN_SCALE)
    h = jnp.dot(h, W2_ref[...],
                preferred_element_type=jnp.float32) + b2_ref[...][None, :]
    out_ref[0] = jnp.max(h, axis=0, keepdims=True)


def _glob_pool(x2pos, mlp_p):
    Bn, n2, din = x2pos.shape
    (W1, b1), (W2, b2) = mlp_p
    out = pl.pallas_call(
        _glob_kernel,
        grid=(Bn,),
        in_specs=[
            pl.BlockSpec((1, n2, din), lambda i: (i, 0, 0)),
            pl.BlockSpec(W1.shape, lambda i: (0, 0)),
            pl.BlockSpec(b1.shape, lambda i: (0,)),
            pl.BlockSpec(W2.shape, lambda i: (0, 0)),
            pl.BlockSpec(b2.shape, lambda i: (0,)),
        ],
        out_specs=pl.BlockSpec((1, 1, W2.shape[1]), lambda i: (i, 0, 0)),
        out_shape=jax.ShapeDtypeStruct((Bn, 1, W2.shape[1]), jnp.float32),
    )(x2pos, W1, b1, W2, b2)
    return out[:, 0, :]


def _interp_mlp_kernel(pd_ref, psT_ref, feats_ref, xskip_ref, *refs,
                       n_layers, fdim):
    wrefs = refs[:-1]
    out_ref = refs[-1]
    pd = pd_ref[0]
    psT = psT_ref[0]
    feats = feats_ref[0]
    d2 = _d2_matrix(pd, psT)
    Q = d2.shape[0]
    num = jnp.zeros((Q, feats.shape[1]), dtype=jnp.float32)
    den = jnp.zeros((Q, 1), dtype=jnp.float32)
    for _ in range(3):
        m = jnp.min(d2, axis=1, keepdims=True)
        oh = d2 == m
        d2 = jnp.where(oh, jnp.inf, d2)
        g = jnp.dot(oh.astype(jnp.float32), feats,
                    preferred_element_type=jnp.float32)
        w = 1.0 / jnp.maximum(m, 1e-16)
        num = num + g * w
        den = den + w
    xi = num / den
    W1 = wrefs[0][...]
    h = (jnp.dot(xi, W1[:fdim, :], preferred_element_type=jnp.float32)
         + jnp.dot(xskip_ref[0], W1[fdim:, :],
                   preferred_element_type=jnp.float32)
         + wrefs[1][...][None, :])
    for i in range(1, n_layers):
        h = jax.nn.relu(h * _BN_SCALE)
        h = jnp.dot(h, wrefs[2 * i][...],
                    preferred_element_type=jnp.float32) + wrefs[2 * i + 1][...][None, :]
    out_ref[0] = h


def _interp_mlp(pd, psT, feats, xskip, mlp_p):
    Bn, Q, _ = pd.shape
    S = psT.shape[2]
    F = feats.shape[2]
    C = xskip.shape[2]
    args = [pd, psT, feats, xskip]
    in_specs = [
        pl.BlockSpec((1, Q, 3), lambda i: (i, 0, 0)),
        pl.BlockSpec((1, 3, S), lambda i: (i, 0, 0)),
        pl.BlockSpec((1, S, F), lambda i: (i, 0, 0)),
        pl.BlockSpec((1, Q, C), lambda i: (i, 0, 0)),
    ]
    for (W, b) in mlp_p:
        args.extend([W, b])
        in_specs.append(pl.BlockSpec(W.shape, lambda i: (0, 0)))
        in_specs.append(pl.BlockSpec(b.shape, lambda i: (0,)))
    dout = mlp_p[-1][0].shape[1]
    return pl.pallas_call(
        functools.partial(_interp_mlp_kernel, n_layers=len(mlp_p), fdim=F),
        grid=(Bn,),
        in_specs=in_specs,
        out_specs=pl.BlockSpec((1, Q, dout), lambda i: (i, 0, 0)),
        out_shape=jax.ShapeDtypeStruct((Bn, Q, dout), jnp.float32),
    )(*args)


def kernel(x, pos, batch, params):
    xb = x.reshape(_B, _P, _ATTR)
    posb = pos.reshape(_B, _P, 3)
    n1, n2 = _P // 4, _P // 16
    idx0, pos1, xdst1, idx1, pos2 = _fps2(posb, xb)
    posT = posb.transpose(0, 2, 1)
    xpos0 = jnp.concatenate([xb, posb], axis=-1)
    x1 = _sa_layer(xpos0, posT, pos1, xdst1, None, _KNN[0],
                   params["sa0_mlp"], params["sa0_lin"])
    pos1T = pos1.transpose(0, 2, 1)
    xpos1 = jnp.concatenate([x1, pos1], axis=-1)
    oh1 = (idx1[:, :, None]
           == jnp.arange(n1, dtype=jnp.int32)[None, None, :]).astype(jnp.float32)
    x2 = _sa_layer(xpos1, pos1T, pos2, oh1, x1, _KNN[1],
                   params["sa1_mlp"], params["sa1_lin"])
    x2pos = jnp.concatenate([x2, pos2], axis=-1)
    xg = _glob_pool(x2pos, params["glob_mlp"])
    fp0_in = jnp.concatenate(
        [jnp.broadcast_to(xg[:, None, :], (_B, n2, 512)), x2], axis=-1)
    h0 = _mlp_pallas(params["fp0_mlp"], fp0_in.reshape(-1, 768)).reshape(_B, n2, 256)
    pos2T = pos2.transpose(0, 2, 1)
    h1 = _interp_mlp(pos1, pos2T, h0, x1, params["fp1_mlp"])
    h2 = _interp_mlp(posb, pos1T, h1, xb, params["fp2_mlp"])
    return h2.reshape(_B * _P, _OUT)
